# baseline (speedup 1.0000x reference)
import jax
import jax.numpy as jnp
from jax.experimental import pallas as pl

_H = 8
_G = 64


def _bilevel_kernel(x_ref, wq_ref, bq_ref, wkv_ref, bkv_ref, wproj_ref,
                    bproj_ref, mask_ref, o_ref):
    C = x_ref.shape[-1]
    rows = x_ref.shape[1]
    x = x_ref[...].reshape(_G * rows, C)
    qpre = jnp.dot(x, wq_ref[...], preferred_element_type=jnp.float32)
    q = jnp.where(qpre + bq_ref[...] >= 2.0, 1.0, 0.0).astype(jnp.bfloat16)
    x16 = jnp.concatenate([x_ref[g, :16] for g in range(_G)], axis=0)
    kvpre = jnp.dot(x16, wkv_ref[...], preferred_element_type=jnp.float32)
    skv = jnp.where(kvpre + bkv_ref[...] >= 2.0, 1.0, 0.0).astype(jnp.bfloat16)
    mask = mask_ref[...]
    outs = []
    for g in range(_G):
        k16 = skv[16 * g:16 * (g + 1), :C]
        v16 = skv[16 * g:16 * (g + 1), C:]
        kv = jnp.dot(k16.T, v16,
                     preferred_element_type=jnp.float32
                     ).astype(jnp.bfloat16) * mask
        ksum = jnp.sum(k16.astype(jnp.float32), axis=0,
                       keepdims=True).astype(jnp.bfloat16)
        qg = q[rows * g:rows * (g + 1)]
        a = jnp.concatenate([kv, ksum.T * mask], axis=1)
        numden = jnp.dot(qg, a, preferred_element_type=jnp.float32)
        outs.append(numden[:, :C] / (numden[:, C:] + 1e-6))
    attn = jnp.concatenate(outs, axis=0).astype(jnp.bfloat16)
    out = jnp.dot(attn, wproj_ref[...], preferred_element_type=jnp.float32)
    o_ref[...] = (out + bproj_ref[...]).reshape(_G, rows, C)


def kernel(x, qkv_w, qkv_b, proj_w, proj_b):
    T, B, Lt, Lh, Lw, C = x.shape
    rows = Lt * Lh * Lw
    dh = C // _H
    x3 = x.reshape(T * B, rows, C)
    wq = qkv_w[:C].T
    bq = qkv_b[:C].reshape(1, C)
    wkv = qkv_w[C:].T
    bkv = qkv_b[C:].reshape(1, 2 * C)
    wproj = proj_w.T.astype(jnp.bfloat16)
    bproj = proj_b.reshape(1, C)
    heads = jnp.arange(C, dtype=jnp.int32) // dh
    mask = (heads[:, None] == heads[None, :]).astype(jnp.bfloat16)

    out = pl.pallas_call(
        _bilevel_kernel,
        grid=(T * B // _G,),
        in_specs=[
            pl.BlockSpec((_G, rows, C), lambda i: (i, 0, 0)),
            pl.BlockSpec((C, C), lambda i: (0, 0)),
            pl.BlockSpec((1, C), lambda i: (0, 0)),
            pl.BlockSpec((C, 2 * C), lambda i: (0, 0)),
            pl.BlockSpec((1, 2 * C), lambda i: (0, 0)),
            pl.BlockSpec((C, C), lambda i: (0, 0)),
            pl.BlockSpec((1, C), lambda i: (0, 0)),
            pl.BlockSpec((C, C), lambda i: (0, 0)),
        ],
        out_specs=pl.BlockSpec((_G, rows, C), lambda i: (i, 0, 0)),
        out_shape=jax.ShapeDtypeStruct((T * B, rows, C), jnp.float32),
    )(x3, wq, bq, wkv, bkv, wproj, bproj, mask)
    return out.reshape(T, B, Lt, Lh, Lw, C)


# speedup vs baseline: 1.0255x; 1.0255x over previous
import jax
import jax.numpy as jnp
from jax.experimental import pallas as pl
from jax.experimental.pallas import tpu as pltpu

_H = 8
_G = 32


def _bilevel_kernel(x_ref, wq_ref, bq_ref, wkv_ref, bkv_ref, wproj_ref,
                    bproj_ref, mask_ref, o_ref):
    C = x_ref.shape[-1]
    rows = x_ref.shape[1]
    x = x_ref[...].reshape(_G * rows, C)
    qpre = jnp.dot(x, wq_ref[...], preferred_element_type=jnp.float32)
    q = jnp.where(qpre + bq_ref[...] >= 2.0, 1.0, 0.0).astype(jnp.bfloat16)
    x16 = jnp.concatenate([x_ref[g, :16] for g in range(_G)], axis=0)
    kvpre = jnp.dot(x16, wkv_ref[...], preferred_element_type=jnp.float32)
    skv = jnp.where(kvpre + bkv_ref[...] >= 2.0, 1.0, 0.0).astype(jnp.bfloat16)
    mask = mask_ref[...]
    outs = []
    for g in range(_G):
        k16 = skv[16 * g:16 * (g + 1), :C]
        v16 = skv[16 * g:16 * (g + 1), C:]
        kv = jnp.dot(k16.T, v16,
                     preferred_element_type=jnp.float32
                     ).astype(jnp.bfloat16) * mask
        ksum = jnp.sum(k16.astype(jnp.float32), axis=0,
                       keepdims=True).astype(jnp.bfloat16)
        qg = q[rows * g:rows * (g + 1)]
        a = jnp.concatenate([kv, ksum.T * mask], axis=1)
        numden = jnp.dot(qg, a, preferred_element_type=jnp.float32)
        outs.append(numden[:, :C] / (numden[:, C:] + 1e-6))
    attn = jnp.concatenate(outs, axis=0).astype(jnp.bfloat16)
    out = jnp.dot(attn, wproj_ref[...], preferred_element_type=jnp.float32)
    o_ref[...] = (out + bproj_ref[...]).reshape(_G, rows, C)


def kernel(x, qkv_w, qkv_b, proj_w, proj_b):
    T, B, Lt, Lh, Lw, C = x.shape
    rows = Lt * Lh * Lw
    dh = C // _H
    x3 = x.reshape(T * B, rows, C)
    wq = qkv_w[:C].T
    bq = qkv_b[:C].reshape(1, C)
    wkv = qkv_w[C:].T
    bkv = qkv_b[C:].reshape(1, 2 * C)
    wproj = proj_w.T.astype(jnp.bfloat16)
    bproj = proj_b.reshape(1, C)
    heads = jnp.arange(C, dtype=jnp.int32) // dh
    mask = (heads[:, None] == heads[None, :]).astype(jnp.bfloat16)

    out = pl.pallas_call(
        _bilevel_kernel,
        grid=(T * B // _G,),
        in_specs=[
            pl.BlockSpec((_G, rows, C), lambda i: (i, 0, 0)),
            pl.BlockSpec((C, C), lambda i: (0, 0)),
            pl.BlockSpec((1, C), lambda i: (0, 0)),
            pl.BlockSpec((C, 2 * C), lambda i: (0, 0)),
            pl.BlockSpec((1, 2 * C), lambda i: (0, 0)),
            pl.BlockSpec((C, C), lambda i: (0, 0)),
            pl.BlockSpec((1, C), lambda i: (0, 0)),
            pl.BlockSpec((C, C), lambda i: (0, 0)),
        ],
        out_specs=pl.BlockSpec((_G, rows, C), lambda i: (i, 0, 0)),
        out_shape=jax.ShapeDtypeStruct((T * B, rows, C), jnp.float32),
        compiler_params=pltpu.CompilerParams(
            dimension_semantics=("parallel",)),
    )(x3, wq, bq, wkv, bkv, wproj, bproj, mask)
    return out.reshape(T, B, Lt, Lh, Lw, C)


# speedup vs baseline: 1.1993x; 1.1695x over previous
import jax
import jax.numpy as jnp
from jax import lax
from jax.experimental import pallas as pl
from jax.experimental.pallas import tpu as pltpu

_H = 8
_G = 32

_DN_T = (((1,), (1,)), ((), ()))


def _bilevel_kernel(x_ref, wqkv_ref, bqkv_ref, wproj_ref, bproj_ref, o_ref):
    C = x_ref.shape[-1]
    rows = x_ref.shape[1]
    dh = C // _H
    x = x_ref[...].reshape(_G * rows, C)
    qpre = lax.dot_general(x, wqkv_ref[:C], _DN_T,
                           preferred_element_type=jnp.float32)
    q = jnp.where(qpre + bqkv_ref[:, :C] >= 2.0, 1.0,
                  0.0).astype(jnp.bfloat16)
    x16 = jnp.concatenate([x_ref[g, :16] for g in range(_G)], axis=0)
    kvpre = lax.dot_general(x16, wqkv_ref[C:], _DN_T,
                            preferred_element_type=jnp.float32)
    skv = jnp.where(kvpre + bqkv_ref[:, C:] >= 2.0, 1.0,
                    0.0).astype(jnp.bfloat16)
    row_h = lax.broadcasted_iota(jnp.int32, (C, C), 0) // dh
    col_h = lax.broadcasted_iota(jnp.int32, (C, C), 1) // dh
    mask = jnp.where(row_h == col_h, 1.0, 0.0).astype(jnp.bfloat16)
    outs = []
    for g in range(_G):
        k16 = skv[16 * g:16 * (g + 1), :C]
        v16 = skv[16 * g:16 * (g + 1), C:]
        kv = jnp.dot(k16.T, v16,
                     preferred_element_type=jnp.float32
                     ).astype(jnp.bfloat16) * mask
        ksum = jnp.sum(k16.astype(jnp.float32), axis=0,
                       keepdims=True).astype(jnp.bfloat16)
        qg = q[rows * g:rows * (g + 1)]
        a = jnp.concatenate([kv, ksum.T * mask], axis=1)
        numden = jnp.dot(qg, a, preferred_element_type=jnp.float32)
        outs.append(numden[:, :C] / (numden[:, C:] + 1e-6))
    attn = jnp.concatenate(outs, axis=0).astype(jnp.bfloat16)
    out = lax.dot_general(attn, wproj_ref[...].astype(jnp.bfloat16), _DN_T,
                          preferred_element_type=jnp.float32)
    o_ref[...] = (out + bproj_ref[...]).reshape(_G, rows, C)


def kernel(x, qkv_w, qkv_b, proj_w, proj_b):
    T, B, Lt, Lh, Lw, C = x.shape
    rows = Lt * Lh * Lw
    x3 = x.reshape(T * B, rows, C)
    bqkv = qkv_b.reshape(1, 3 * C)
    bproj = proj_b.reshape(1, C)

    out = pl.pallas_call(
        _bilevel_kernel,
        grid=(T * B // _G,),
        in_specs=[
            pl.BlockSpec((_G, rows, C), lambda i: (i, 0, 0)),
            pl.BlockSpec((3 * C, C), lambda i: (0, 0)),
            pl.BlockSpec((1, 3 * C), lambda i: (0, 0)),
            pl.BlockSpec((C, C), lambda i: (0, 0)),
            pl.BlockSpec((1, C), lambda i: (0, 0)),
        ],
        out_specs=pl.BlockSpec((_G, rows, C), lambda i: (i, 0, 0)),
        out_shape=jax.ShapeDtypeStruct((T * B, rows, C), jnp.float32),
        compiler_params=pltpu.CompilerParams(
            dimension_semantics=("parallel",)),
    )(x3, qkv_w, bqkv, proj_w, bproj)
    return out.reshape(T, B, Lt, Lh, Lw, C)
